# Initial kernel scaffold; baseline (speedup 1.0000x reference)
import jax
import jax.numpy as jnp
from jax.experimental import pallas as pl


def kernel(x, r_double, f_double, r_ij, r_ik, triple_ijk, neighbor_mask, triple_mask, W_in2f, Wd1, bd1, Wd2, bd2, Wt1, bt1, Wt2, bt2, Wout, bout, neighbors, neighbors_j, neighbors_k):
    raise NotImplementedError("write your pallas kernel here")


# speedup vs baseline: 23.6250x; 23.6250x over previous
import functools
import jax
import jax.numpy as jnp
from jax import lax
from jax.experimental import pallas as pl

B, At, Nd, Nt = 8, 128, 32, 96
N_IN, N_FILTERS, N_OUT = 128, 128, 128
NG, NA = 25, 20

AT_TILES = 4
TA = At // AT_TILES


def _ssp(v):
    return jax.nn.softplus(v) - jnp.log(2.0)


def _body(x_ref, fd_ref, ft_ref, nbr_ref, nbj_ref, nbk_ref, nm_ref, tm_ref,
          w_in2f_ref, wd1_ref, bd1_ref, wd2_ref, bd2_ref,
          wt1_ref, bt1_ref, wt2_ref, bt2_ref, wout_ref, bout_ref, out_ref):
    f32 = jnp.float32
    y = jnp.dot(x_ref[0], w_in2f_ref[...], preferred_element_type=f32)

    nd = TA * Nd
    nt = TA * Nt

    fd = fd_ref[0]
    h = _ssp(jnp.dot(fd, wd1_ref[...], preferred_element_type=f32) + bd1_ref[...])
    w_double = jnp.dot(h, wd2_ref[...], preferred_element_type=f32) + bd2_ref[...]

    idx_d = nbr_ref[0]
    iota_d = lax.broadcasted_iota(jnp.int32, (At, nd), 0)
    onehot_d = jnp.where(iota_d == idx_d, f32(1.0), f32(0.0)) * nm_ref[0]
    g_d = lax.dot_general(onehot_d, y, (((0,), (0,)), ((), ())),
                          preferred_element_type=f32)
    yd = (g_d * w_double).reshape(TA, Nd, N_FILTERS).sum(axis=1)

    ft = ft_ref[0]
    h_t = _ssp(jnp.dot(ft, wt1_ref[...], preferred_element_type=f32) + bt1_ref[...])
    w_triple = jnp.dot(h_t, wt2_ref[...], preferred_element_type=f32) + bt2_ref[...]

    idx_j = nbj_ref[0]
    idx_k = nbk_ref[0]
    iota_t = lax.broadcasted_iota(jnp.int32, (At, nt), 0)
    p_t = (jnp.where(iota_t == idx_j, f32(1.0), f32(0.0)) +
           jnp.where(iota_t == idx_k, f32(1.0), f32(0.0))) * tm_ref[0]
    g_t = lax.dot_general(p_t, y, (((0,), (0,)), ((), ())),
                          preferred_element_type=f32)
    yt = (g_t * w_triple).reshape(TA, Nt, N_FILTERS).sum(axis=1)

    cat = jnp.concatenate((yd, yt), axis=1)
    out_ref[0] = jnp.dot(cat, wout_ref[...], preferred_element_type=f32) + bout_ref[...]


def kernel(x, r_double, f_double, r_ij, r_ik, triple_ijk, neighbor_mask,
           triple_mask, W_in2f, Wd1, bd1, Wd2, bd2, Wt1, bt1, Wt2, bt2,
           Wout, bout, neighbors, neighbors_j, neighbors_k):
    nd = TA * Nd
    nt = TA * Nt
    fd = f_double.reshape(B * AT_TILES, nd, NG)
    ft = triple_ijk.reshape(B * AT_TILES, nt, NA)
    nbr = neighbors.reshape(B * AT_TILES, 1, nd)
    nbj = neighbors_j.reshape(B * AT_TILES, 1, nt)
    nbk = neighbors_k.reshape(B * AT_TILES, 1, nt)
    nm = neighbor_mask.reshape(B * AT_TILES, 1, nd)
    tm = triple_mask.reshape(B * AT_TILES, 1, nt)
    bd1_ = bd1.reshape(1, N_FILTERS)
    bd2_ = bd2.reshape(1, N_FILTERS)
    bt1_ = bt1.reshape(1, N_FILTERS)
    bt2_ = bt2.reshape(1, N_FILTERS)
    bout_ = bout.reshape(1, N_OUT)

    tile_map = lambda b, t: (b * AT_TILES + t, 0, 0)
    full2 = lambda shape: pl.BlockSpec(shape, lambda b, t: (0, 0))

    out = pl.pallas_call(
        _body,
        grid=(B, AT_TILES),
        in_specs=[
            pl.BlockSpec((1, At, N_IN), lambda b, t: (b, 0, 0)),
            pl.BlockSpec((1, nd, NG), tile_map),
            pl.BlockSpec((1, nt, NA), tile_map),
            pl.BlockSpec((1, 1, nd), tile_map),
            pl.BlockSpec((1, 1, nt), tile_map),
            pl.BlockSpec((1, 1, nt), tile_map),
            pl.BlockSpec((1, 1, nd), tile_map),
            pl.BlockSpec((1, 1, nt), tile_map),
            full2((N_IN, N_FILTERS)),
            full2((NG, N_FILTERS)),
            full2((1, N_FILTERS)),
            full2((N_FILTERS, N_FILTERS)),
            full2((1, N_FILTERS)),
            full2((NA, N_FILTERS)),
            full2((1, N_FILTERS)),
            full2((N_FILTERS, N_FILTERS)),
            full2((1, N_FILTERS)),
            full2((2 * N_FILTERS, N_OUT)),
            full2((1, N_OUT)),
        ],
        out_specs=pl.BlockSpec((1, TA, N_OUT), lambda b, t: (b, t, 0)),
        out_shape=jax.ShapeDtypeStruct((B, At, N_OUT), jnp.float32),
    )(x, fd, ft, nbr, nbj, nbk, nm, tm, W_in2f, Wd1, bd1_, Wd2, bd2_,
      Wt1, bt1_, Wt2, bt2_, Wout, bout_)
    return out
